# Initial kernel scaffold; baseline (speedup 1.0000x reference)
import jax
import jax.numpy as jnp
from jax.experimental import pallas as pl


def kernel(x, edge_index, basis, comp, root, bias):
    raise NotImplementedError("write your pallas kernel here")


# speedup vs baseline: 5.5802x; 5.5802x over previous
import functools

import jax
import jax.numpy as jnp
from jax import lax
from jax.experimental import pallas as pl
from jax.experimental.pallas import tpu as pltpu
from jax.experimental.pallas import tpu_sc as plsc

N = 10000
E = 320000
D = 128

_NC = 2
_NS = 16
_CHUNK = 80
_EDGES_PER_TILE = E // (_NC * _NS)
_CHUNKS_PER_TILE = _EDGES_PER_TILE // _CHUNK
_ROW_CHUNKS = N // _CHUNK


def _sc_aggregate(src, dst, x):
    mesh = plsc.VectorSubcoreMesh(core_axis_name="c", subcore_axis_name="s")

    @functools.partial(
        pl.kernel,
        mesh=mesh,
        out_type=jax.ShapeDtypeStruct((_NC, N, D), jnp.float32),
        scratch_types=[
            pltpu.VMEM((_CHUNK,), jnp.int32),
            pltpu.VMEM((_CHUNK,), jnp.int32),
            pltpu.VMEM((_CHUNK, D), jnp.float32),
            pltpu.VMEM_SHARED((N, D), jnp.float32),
            pltpu.SemaphoreType.DMA,
        ],
    )
    def agg_kernel(src_hbm, dst_hbm, x_hbm, out_hbm, src_v, dst_v, rows_v, acc_sh, sem):
        c = lax.axis_index("c")
        s = lax.axis_index("s")

        zeros16 = jnp.zeros((16,), jnp.float32)

        def zero_row(i, _):
            for j in range(D // 16):
                rows_v[i, pl.ds(j * 16, 16)] = zeros16
            return 0

        lax.fori_loop(0, _CHUNK, zero_row, 0)

        n_row_chunks = jnp.where(s < _ROW_CHUNKS - _NS * (_ROW_CHUNKS // _NS),
                                 _ROW_CHUNKS // _NS + 1, _ROW_CHUNKS // _NS)

        def zero_chunk(k, _):
            off = pl.multiple_of((s + k * _NS) * _CHUNK, 8)
            pltpu.sync_copy(rows_v, acc_sh.at[pl.ds(off, _CHUNK)])
            return 0

        lax.fori_loop(0, n_row_chunks, zero_chunk, 0)

        plsc.subcore_barrier()

        base = (c * _NS + s) * _EDGES_PER_TILE

        def body(j, _):
            off = pl.multiple_of(base + j * _CHUNK, 8)
            pltpu.sync_copy(src_hbm.at[pl.ds(off, _CHUNK)], src_v)
            pltpu.sync_copy(dst_hbm.at[pl.ds(off, _CHUNK)], dst_v)
            pltpu.async_copy(x_hbm.at[src_v], rows_v, sem).wait()
            pltpu.sync_copy(rows_v, acc_sh.at[dst_v], add=True)
            return 0

        lax.fori_loop(0, _CHUNKS_PER_TILE, body, 0)

        plsc.subcore_barrier()

        def flush_chunk(k, _):
            off = pl.multiple_of((s + k * _NS) * _CHUNK, 8)
            pltpu.sync_copy(acc_sh.at[pl.ds(off, _CHUNK)],
                            out_hbm.at[c, pl.ds(off, _CHUNK)])
            return 0

        lax.fori_loop(0, n_row_chunks, flush_chunk, 0)

    return agg_kernel(src, dst, x)


def _finish_body(p_ref, x_ref, basis_ref, comp_ref, root_ref, bias_ref, o_ref):
    agg = p_ref[0] + p_ref[1]
    w_msg = basis_ref[...] * comp_ref[0, 0]
    o_ref[...] = (
        jnp.dot(agg, w_msg, preferred_element_type=jnp.float32)
        + jnp.dot(x_ref[...], root_ref[...], preferred_element_type=jnp.float32)
        + bias_ref[...]
    )


def _tc_finish(partials, x, basis0, comp, root, bias):
    blk = 1000
    grid = (N // blk,)
    return pl.pallas_call(
        _finish_body,
        grid=grid,
        in_specs=[
            pl.BlockSpec((_NC, blk, D), lambda i: (0, i, 0)),
            pl.BlockSpec((blk, D), lambda i: (i, 0)),
            pl.BlockSpec((D, D), lambda i: (0, 0)),
            pl.BlockSpec(memory_space=pltpu.SMEM),
            pl.BlockSpec((D, D), lambda i: (0, 0)),
            pl.BlockSpec((1, D), lambda i: (0, 0)),
        ],
        out_specs=pl.BlockSpec((blk, D), lambda i: (i, 0)),
        out_shape=jax.ShapeDtypeStruct((N, D), jnp.float32),
    )(partials, x, basis0, comp, root, bias)


def kernel(x, edge_index, basis, comp, root, bias):
    src = edge_index[0].astype(jnp.int32)
    dst = edge_index[1].astype(jnp.int32)
    partials = _sc_aggregate(src, dst, x)
    return _tc_finish(partials, x, basis[0], comp.reshape(1, 1),
                      root, bias.reshape(1, D))


# speedup vs baseline: 13.6861x; 2.4526x over previous
import functools

import jax
import jax.numpy as jnp
from jax import lax
from jax.experimental import pallas as pl
from jax.experimental.pallas import tpu as pltpu
from jax.experimental.pallas import tpu_sc as plsc

N = 10000
E = 320000
D = 128

_NC = 2
_NS = 16
_CHUNK = 80
_EDGES_PER_TILE = E // (_NC * _NS)
_CHUNKS_PER_TILE = _EDGES_PER_TILE // _CHUNK
_ROW_CHUNKS = N // _CHUNK


_NBUF = 4


def _sc_aggregate(src, dst, x):
    mesh = plsc.VectorSubcoreMesh(core_axis_name="c", subcore_axis_name="s")

    @functools.partial(
        pl.kernel,
        mesh=mesh,
        out_type=jax.ShapeDtypeStruct((_NC, N, D), jnp.float32),
        scratch_types=[pltpu.VMEM((_CHUNK,), jnp.int32) for _ in range(_NBUF)]
        + [pltpu.VMEM((_CHUNK,), jnp.int32) for _ in range(_NBUF)]
        + [pltpu.VMEM((_CHUNK, D), jnp.float32) for _ in range(_NBUF)]
        + [pltpu.VMEM_SHARED((N, D), jnp.float32)]
        + [pltpu.SemaphoreType.DMA for _ in range(4 * _NBUF)],
    )
    def agg_kernel(src_hbm, dst_hbm, x_hbm, out_hbm,
                   si0, si1, si2, si3, di0, di1, di2, di3,
                   r0, r1, r2, r3, acc_sh,
                   i0, i1, i2, i3, j0, j1, j2, j3,
                   g0, g1, g2, g3, s0, s1, s2, s3):
        sidx = (si0, si1, si2, si3)
        didx = (di0, di1, di2, di3)
        rows = (r0, r1, r2, r3)
        isem = (i0, i1, i2, i3)
        jsem = (j0, j1, j2, j3)
        gsem = (g0, g1, g2, g3)
        ssem = (s0, s1, s2, s3)
        c = lax.axis_index("c")
        s = lax.axis_index("s")
        base = (c * _NS + s) * _EDGES_PER_TILE

        def fire_idx(j, b):
            off = pl.multiple_of(base + j * _CHUNK, 8)
            pltpu.async_copy(src_hbm.at[pl.ds(off, _CHUNK)], sidx[b], isem[b])
            pltpu.async_copy(dst_hbm.at[pl.ds(off, _CHUNK)], didx[b], jsem[b])

        def wait_idx(b):
            pltpu.make_async_copy(src_hbm.at[pl.ds(0, _CHUNK)], sidx[b], isem[b]).wait()
            pltpu.make_async_copy(dst_hbm.at[pl.ds(0, _CHUNK)], didx[b], jsem[b]).wait()

        def fire_gather(b):
            pltpu.async_copy(x_hbm.at[sidx[b]], rows[b], gsem[b])

        def wait_gather(b):
            pltpu.make_async_copy(x_hbm.at[sidx[b]], rows[b], gsem[b]).wait()

        def scat(b):
            pltpu.async_copy(rows[b], acc_sh.at[didx[b]], ssem[b], add=True).wait()

        for b in range(_NBUF):
            fire_idx(b, b)

        zeros16 = jnp.zeros((16,), jnp.float32)

        def zero_row(i, _):
            for j in range(D // 16):
                r0[i, pl.ds(j * 16, 16)] = zeros16
            return 0

        lax.fori_loop(0, _CHUNK, zero_row, 0)

        n_row_chunks = jnp.where(s < _ROW_CHUNKS - _NS * (_ROW_CHUNKS // _NS),
                                 _ROW_CHUNKS // _NS + 1, _ROW_CHUNKS // _NS)

        def zero_chunk(k, _):
            off = pl.multiple_of((s + k * _NS) * _CHUNK, 8)
            pltpu.sync_copy(r0, acc_sh.at[pl.ds(off, _CHUNK)])
            return 0

        lax.fori_loop(0, n_row_chunks, zero_chunk, 0)
        plsc.subcore_barrier()

        for b in range(_NBUF - 1):
            wait_idx(b)
            fire_gather(b)

        def step(j, b, bg, fire_i, fire_g):
            wait_gather(b)
            scat(b)
            if fire_i:
                fire_idx(j + _NBUF, b)
            if fire_g:
                wait_idx(bg)
                fire_gather(bg)

        def body(i, _):
            for b in range(_NBUF):
                j = i * _NBUF + b
                step(j, b, (b + _NBUF - 1) % _NBUF, True, True)
            return 0

        n_main = (_CHUNKS_PER_TILE - _NBUF) // _NBUF
        lax.fori_loop(0, n_main, body, 0)

        for j in range(_NBUF * n_main, _CHUNKS_PER_TILE):
            step(j, j % _NBUF, (j + _NBUF - 1) % _NBUF,
                 j + _NBUF < _CHUNKS_PER_TILE,
                 j + _NBUF - 1 < _CHUNKS_PER_TILE)
        plsc.subcore_barrier()

        def flush_chunk(k, _):
            off = pl.multiple_of((s + k * _NS) * _CHUNK, 8)
            pltpu.sync_copy(acc_sh.at[pl.ds(off, _CHUNK)],
                            out_hbm.at[c, pl.ds(off, _CHUNK)])
            return 0

        lax.fori_loop(0, n_row_chunks, flush_chunk, 0)

    return agg_kernel(src, dst, x)


def _finish_body(p_ref, x_ref, basis_ref, comp_ref, root_ref, bias_ref, o_ref):
    agg = p_ref[0] + p_ref[1]
    w_msg = basis_ref[...] * comp_ref[0, 0]
    o_ref[...] = (
        jnp.dot(agg, w_msg, preferred_element_type=jnp.float32,
                precision=jax.lax.Precision.HIGHEST)
        + jnp.dot(x_ref[...], root_ref[...], preferred_element_type=jnp.float32,
                  precision=jax.lax.Precision.HIGHEST)
        + bias_ref[...]
    )


def _tc_finish(partials, x, basis0, comp, root, bias):
    blk = 1000
    grid = (N // blk,)
    return pl.pallas_call(
        _finish_body,
        grid=grid,
        in_specs=[
            pl.BlockSpec((_NC, blk, D), lambda i: (0, i, 0)),
            pl.BlockSpec((blk, D), lambda i: (i, 0)),
            pl.BlockSpec((D, D), lambda i: (0, 0)),
            pl.BlockSpec(memory_space=pltpu.SMEM),
            pl.BlockSpec((D, D), lambda i: (0, 0)),
            pl.BlockSpec((1, D), lambda i: (0, 0)),
        ],
        out_specs=pl.BlockSpec((blk, D), lambda i: (i, 0)),
        out_shape=jax.ShapeDtypeStruct((N, D), jnp.float32),
    )(partials, x, basis0, comp, root, bias)


def kernel(x, edge_index, basis, comp, root, bias):
    src = edge_index[0].astype(jnp.int32)
    dst = edge_index[1].astype(jnp.int32)
    partials = _sc_aggregate(src, dst, x)
    return _tc_finish(partials, x, basis[0], comp.reshape(1, 1),
                      root, bias.reshape(1, D))
